# baseline (speedup 1.0000x reference)
import functools

import jax
import jax.numpy as jnp
from jax import lax
from jax.experimental import pallas as pl
from jax.experimental.pallas import tpu as pltpu
from jax.experimental.pallas import tpu_sc as plsc

_B, _N, _D = 4, 8192, 768
_BT = _B * _N
_NW = 32
_BPW = _BT // _NW
_C = 64
_NCHUNK = _BPW // _C

_CH = 1024
_NBLK = _BT // _CH
_NPB = _N // _CH
_TARGET_RATIOS = (0.1, 0.02)


def _sc_gather_kernel(table_hbm, idx_hbm, out_hbm, idx_v, buf, sem0, sem1):
    wid = lax.axis_index("s") * 2 + lax.axis_index("c")
    base = wid * _BPW
    pltpu.sync_copy(idx_hbm.at[pl.ds(base, _BPW)], idx_v)
    sems = (sem0, sem1)

    for t in range(2):
        pltpu.async_copy(
            table_hbm.at[idx_v.at[pl.ds(t * _C, _C)]], buf.at[t], sems[t]
        )

    def body(i, carry):
        for t in range(2):
            c = i * 2 + t
            pltpu.make_async_copy(
                table_hbm.at[pl.ds(0, _C)], buf.at[t], sems[t]
            ).wait()
            pltpu.sync_copy(buf.at[t], out_hbm.at[pl.ds(base + c * _C, _C)])
            nxt = c + 2

            @pl.when(nxt < _NCHUNK)
            def _():
                pltpu.async_copy(
                    table_hbm.at[idx_v.at[pl.ds(nxt * _C, _C)]],
                    buf.at[t],
                    sems[t],
                )
        return carry

    lax.fori_loop(0, _NCHUNK // 2, body, 0)


@functools.cache
def _sc_gather():
    return pl.kernel(
        _sc_gather_kernel,
        out_type=jax.ShapeDtypeStruct((_BT, _D), jnp.float32),
        mesh=plsc.VectorSubcoreMesh(core_axis_name="c", subcore_axis_name="s"),
        scratch_types=[
            pltpu.VMEM((_BPW,), jnp.int32),
            pltpu.VMEM((2, _C, _D), jnp.float32),
            pltpu.SemaphoreType.DMA,
            pltpu.SemaphoreType.DMA,
        ],
    )


def _tc_router_body(
    x_ref, m_ref, w_ref, b_ref, tri_ref,
    lid1_ref, hd1_ref, lid2_ref, hd2_ref, loss_ref,
    carry_ref, acc_ref,
):
    i = pl.program_id(0)

    @pl.when(i == 0)
    def _():
        acc_ref[0] = 0.0
        acc_ref[1] = 0.0
        acc_ref[2] = 0.0

    @pl.when(i % _NPB == 0)
    def _():
        carry_ref[0] = 0
        carry_ref[1] = 0

    x = x_ref[...]
    logits = jnp.dot(x, w_ref[...], preferred_element_type=jnp.float32)
    maskf = m_ref[...].astype(jnp.float32)

    l1 = logits[:, 0:1] + b_ref[0]
    l2 = logits[:, 1:2] + b_ref[1]
    p1 = jax.nn.sigmoid(l1) * maskf
    p2 = jax.nn.sigmoid(l2) * maskf
    h1 = (p1 > 0.5).astype(jnp.float32)
    h2 = (p2 > 0.5).astype(jnp.float32)

    hh = jnp.concatenate([h1, h2], axis=1)
    cs = jnp.dot(tri_ref[...], hh, preferred_element_type=jnp.float32)
    cs1 = cs[:, 0:1].astype(jnp.int32) + carry_ref[0]
    cs2 = cs[:, 1:2].astype(jnp.int32) + carry_ref[1]
    lid1_ref[...] = cs1
    hd1_ref[...] = h1.astype(jnp.int32)
    lid2_ref[...] = cs2
    hd2_ref[...] = h2.astype(jnp.int32)
    carry_ref[0] = cs1[_CH - 1, 0]
    carry_ref[1] = cs2[_CH - 1, 0]

    acc_ref[0] += jnp.sum(p1)
    acc_ref[1] += jnp.sum(p2)
    acc_ref[2] += jnp.sum(maskf)

    @pl.when(i == _NBLK - 1)
    def _():
        denom = jnp.maximum(acc_ref[2], 1.0)
        r1 = acc_ref[0] / denom
        r2 = acc_ref[1] / denom
        loss_ref[0, 0] = (
            (r1 - _TARGET_RATIOS[0]) ** 2 + (r2 - _TARGET_RATIOS[1]) ** 2
        )


_TC_GRID = (_NBLK,)
_TC_IN_SPECS = [
    pl.BlockSpec((_CH, _D), lambda i: (i, 0)),
    pl.BlockSpec((_CH, 1), lambda i: (i, 0)),
    pl.BlockSpec((_D, 2), lambda i: (0, 0)),
    pl.BlockSpec(memory_space=pltpu.SMEM),
    pl.BlockSpec((_CH, _CH), lambda i: (0, 0)),
]
_TC_OUT_SPECS = [
    pl.BlockSpec((_CH, 1), lambda i: (i, 0)),
    pl.BlockSpec((_CH, 1), lambda i: (i, 0)),
    pl.BlockSpec((_CH, 1), lambda i: (i, 0)),
    pl.BlockSpec((_CH, 1), lambda i: (i, 0)),
    pl.BlockSpec(memory_space=pltpu.SMEM),
]
_TC_OUT_SHAPES = [
    jax.ShapeDtypeStruct((_BT, 1), jnp.int32),
    jax.ShapeDtypeStruct((_BT, 1), jnp.int32),
    jax.ShapeDtypeStruct((_BT, 1), jnp.int32),
    jax.ShapeDtypeStruct((_BT, 1), jnp.int32),
    jax.ShapeDtypeStruct((1, 1), jnp.float32),
]
_TC_SCRATCH = [pltpu.SMEM((2,), jnp.int32), pltpu.SMEM((4,), jnp.float32)]


def _tc_router(x2, m2, wc, bc):
    tri = jnp.tril(jnp.ones((_CH, _CH), jnp.float32))
    return pl.pallas_call(
        _tc_router_body,
        grid=_TC_GRID,
        in_specs=_TC_IN_SPECS,
        out_specs=_TC_OUT_SPECS,
        out_shape=_TC_OUT_SHAPES,
        scratch_shapes=_TC_SCRATCH,
    )(x2, m2, wc, bc, tri)


def kernel(input_ids, attention_mask, token_embed, W_r1, b_r1, W_r2, b_r2):
    B, N = input_ids.shape
    V, D = token_embed.shape
    assert (B, N, D) == (_B, _N, _D)

    idx = input_ids.reshape(_BT)
    x2 = _sc_gather()(token_embed, idx)

    m2 = attention_mask.reshape(_BT, 1)
    wc = jnp.concatenate([W_r1, W_r2], axis=1)
    bc = jnp.concatenate([b_r1, b_r2])
    lid1, hd1, lid2, hd2, loss = _tc_router(x2, m2, wc, bc)

    x = x2.reshape(B, N, D)
    return (
        x,
        lid1.reshape(B, N),
        hd1.reshape(B, N).astype(bool),
        lid2.reshape(B, N),
        hd2.reshape(B, N).astype(bool),
        loss[0, 0],
    )


# speedup vs baseline: 1.2670x; 1.2670x over previous
import functools

import jax
import jax.numpy as jnp
from jax import lax
from jax.experimental import pallas as pl
from jax.experimental.pallas import tpu as pltpu
from jax.experimental.pallas import tpu_sc as plsc

_B, _N, _D = 4, 8192, 768
_BT = _B * _N
_NW = 32
_BPW = _BT // _NW
_C = 64
_NCHUNK = _BPW // _C

_CH = 1024
_NBLK = _BT // _CH
_R, _CC = 256, 128
_RPB = _N // _CC
_TARGET_RATIOS = (0.1, 0.02)


def _sc_gather_kernel(table_hbm, idx_hbm, out_hbm, idx_v, buf, sem0, sem1):
    wid = lax.axis_index("s") * 2 + lax.axis_index("c")
    base = wid * _BPW
    pltpu.sync_copy(idx_hbm.at[pl.ds(base, _BPW)], idx_v)
    sems = (sem0, sem1)

    for t in range(2):
        pltpu.async_copy(
            table_hbm.at[idx_v.at[pl.ds(t * _C, _C)]], buf.at[t], sems[t]
        )

    def body(i, carry):
        for t in range(2):
            c = i * 2 + t
            pltpu.make_async_copy(
                table_hbm.at[pl.ds(0, _C)], buf.at[t], sems[t]
            ).wait()
            pltpu.sync_copy(buf.at[t], out_hbm.at[pl.ds(base + c * _C, _C)])
            nxt = c + 2

            @pl.when(nxt < _NCHUNK)
            def _():
                pltpu.async_copy(
                    table_hbm.at[idx_v.at[pl.ds(nxt * _C, _C)]],
                    buf.at[t],
                    sems[t],
                )
        return carry

    lax.fori_loop(0, _NCHUNK // 2, body, 0)


@functools.cache
def _sc_gather():
    return pl.kernel(
        _sc_gather_kernel,
        out_type=jax.ShapeDtypeStruct((_BT, _D), jnp.float32),
        mesh=plsc.VectorSubcoreMesh(core_axis_name="c", subcore_axis_name="s"),
        scratch_types=[
            pltpu.VMEM((_BPW,), jnp.int32),
            pltpu.VMEM((2, _C, _D), jnp.float32),
            pltpu.SemaphoreType.DMA,
            pltpu.SemaphoreType.DMA,
        ],
    )


def _tc_logits_body(x_ref, w_ref, l1_ref, l2_ref):
    lg = jnp.dot(x_ref[...], w_ref[...], preferred_element_type=jnp.float32)
    l1_ref[...] = lg[:, 0:1]
    l2_ref[...] = lg[:, 1:2]


def _tc_logits(x2, wc):
    return pl.pallas_call(
        _tc_logits_body,
        grid=(_NBLK,),
        in_specs=[
            pl.BlockSpec((_CH, _D), lambda i: (i, 0)),
            pl.BlockSpec((_D, 2), lambda i: (0, 0)),
        ],
        out_specs=[
            pl.BlockSpec((_CH, 1), lambda i: (i, 0)),
            pl.BlockSpec((_CH, 1), lambda i: (i, 0)),
        ],
        out_shape=[
            jax.ShapeDtypeStruct((_BT, 1), jnp.float32),
            jax.ShapeDtypeStruct((_BT, 1), jnp.float32),
        ],
    )(x2, wc)


def _tc_finish_body(
    l1_ref, l2_ref, m_ref, b_ref,
    lid1_ref, hd1_ref, lid2_ref, hd2_ref, loss_ref,
):
    l1 = l1_ref[...] + b_ref[0]
    l2 = l2_ref[...] + b_ref[1]
    maskf = m_ref[...].astype(jnp.float32)
    p1 = jax.nn.sigmoid(l1) * maskf
    p2 = jax.nn.sigmoid(l2) * maskf
    h1 = (p1 > 0.5).astype(jnp.float32)
    h2 = (p2 > 0.5).astype(jnp.float32)

    rowc = lax.broadcasted_iota(jnp.int32, (_CC, _CC), 0)
    colc = lax.broadcasted_iota(jnp.int32, (_CC, _CC), 1)
    upper = (rowc <= colc).astype(jnp.float32)
    win1 = jnp.dot(h1, upper, preferred_element_type=jnp.float32)
    win2 = jnp.dot(h2, upper, preferred_element_type=jnp.float32)

    rowr = lax.broadcasted_iota(jnp.int32, (_R, _R), 0)
    colr = lax.broadcasted_iota(jnp.int32, (_R, _R), 1)
    batch_start = (rowr // _RPB) * _RPB
    wb_lower = jnp.logical_and(colr < rowr, colr >= batch_start)
    wb_lower = wb_lower.astype(jnp.float32)
    rs1 = win1[:, _CC - 1 : _CC]
    rs2 = win2[:, _CC - 1 : _CC]
    cs1 = win1 + jnp.dot(wb_lower, rs1, preferred_element_type=jnp.float32)
    cs2 = win2 + jnp.dot(wb_lower, rs2, preferred_element_type=jnp.float32)

    lid1_ref[...] = cs1.astype(jnp.int32)
    hd1_ref[...] = h1.astype(jnp.int32)
    lid2_ref[...] = cs2.astype(jnp.int32)
    hd2_ref[...] = h2.astype(jnp.int32)

    denom = jnp.maximum(jnp.sum(maskf), 1.0)
    r1 = jnp.sum(p1) / denom
    r2 = jnp.sum(p2) / denom
    loss_ref[0, 0] = (
        (r1 - _TARGET_RATIOS[0]) ** 2 + (r2 - _TARGET_RATIOS[1]) ** 2
    )


def _tc_finish(l1, l2, mv, bc):
    return pl.pallas_call(
        _tc_finish_body,
        in_specs=[
            pl.BlockSpec((_R, _CC), lambda: (0, 0)),
            pl.BlockSpec((_R, _CC), lambda: (0, 0)),
            pl.BlockSpec((_R, _CC), lambda: (0, 0)),
            pl.BlockSpec(memory_space=pltpu.SMEM),
        ],
        out_specs=[
            pl.BlockSpec((_R, _CC), lambda: (0, 0)),
            pl.BlockSpec((_R, _CC), lambda: (0, 0)),
            pl.BlockSpec((_R, _CC), lambda: (0, 0)),
            pl.BlockSpec((_R, _CC), lambda: (0, 0)),
            pl.BlockSpec(memory_space=pltpu.SMEM),
        ],
        out_shape=[
            jax.ShapeDtypeStruct((_R, _CC), jnp.int32),
            jax.ShapeDtypeStruct((_R, _CC), jnp.int32),
            jax.ShapeDtypeStruct((_R, _CC), jnp.int32),
            jax.ShapeDtypeStruct((_R, _CC), jnp.int32),
            jax.ShapeDtypeStruct((1, 1), jnp.float32),
        ],
    )(l1, l2, mv, bc)


def kernel(input_ids, attention_mask, token_embed, W_r1, b_r1, W_r2, b_r2):
    B, N = input_ids.shape
    V, D = token_embed.shape
    assert (B, N, D) == (_B, _N, _D)

    idx = input_ids.reshape(_BT)
    x2 = _sc_gather()(token_embed, idx)

    wc = jnp.concatenate([W_r1, W_r2], axis=1)
    bc = jnp.concatenate([b_r1, b_r2])
    l1, l2 = _tc_logits(x2, wc)
    mv = attention_mask.reshape(_R, _CC)
    lid1, hd1, lid2, hd2, loss = _tc_finish(
        l1.reshape(_R, _CC), l2.reshape(_R, _CC), mv, bc
    )

    x = x2.reshape(B, N, D)
    return (
        x,
        lid1.reshape(B, N),
        hd1.reshape(B, N).astype(bool),
        lid2.reshape(B, N),
        hd2.reshape(B, N).astype(bool),
        loss[0, 0],
    )
